# baseline (speedup 1.0000x reference)
import functools

import jax
import jax.numpy as jnp
from jax import lax
from jax.experimental import pallas as pl
from jax.experimental.pallas import tpu as pltpu
from jax.experimental.pallas import tpu_sc as plsc

_N = 4096
_M = 20
_K = 1000
_CI = 128
_NCHUNK = _M * (_N // _CI)
_SPLITS = ((0, 256), (256, 256), (512, 256), (768, 232))


def _make_sc_kernel():
    info = plsc.get_sparse_core_info()
    nc, ns = info.num_cores, info.num_subcores
    nw = nc * ns
    cpw = _NCHUNK // nw
    mesh = plsc.VectorSubcoreMesh(core_axis_name="c", subcore_axis_name="s")

    @functools.partial(
        pl.kernel,
        mesh=mesh,
        compiler_params=pltpu.CompilerParams(needs_layout_passes=False),
        out_type=jax.ShapeDtypeStruct((_M * _K, _N), jnp.float32),
        scratch_types=[
            *[pltpu.VMEM((sz, _CI), jnp.float32) for _, sz in _SPLITS],
            pltpu.VMEM((_NCHUNK // 32 * _CI,), jnp.int32),
            *[pltpu.SemaphoreType.DMA for _ in _SPLITS],
        ],
    )
    def sc_onehot(idx_hbm, out_hbm, buf0, buf1, buf2, buf3,
                  idx_v, sem0, sem1, sem2, sem3):
        wid = lax.axis_index("s") * nc + lax.axis_index("c")
        lanes = lax.iota(jnp.int32, 16)
        ones = jnp.ones((16,), jnp.float32)
        zeros = jnp.zeros((16,), jnp.float32)
        bufs = (buf0, buf1, buf2, buf3)
        sems = (sem0, sem1, sem2, sem3)
        halves = tuple(
            (bufs[x], sems[x], _SPLITS[x][0], _SPLITS[x][1])
            for x in range(len(_SPLITS)))

        pltpu.sync_copy(idx_hbm.at[pl.ds(wid * cpw * _CI, cpw * _CI)], idx_v)

        for _buf, _sz in zip(bufs, [sz for _, sz in _SPLITS]):
            def zero_body(z, _, _buf=_buf):
                cell = lanes + z * 16
                plsc.store_scatter(_buf, [cell // _CI, cell % _CI], zeros)
                return _

            lax.fori_loop(0, _sz * _CI // 16, zero_body, None, unroll=8)

        def scatter(buf, c, base, size, vals):
            def grp(g, _):
                ids = idx_v[pl.ds(c * _CI + g * 16, 16)]
                rel = ids - base
                msk = jnp.logical_and(rel >= 0, rel < size)
                rel = jnp.where(msk, rel, 0)
                plsc.store_scatter(
                    buf, [rel, lanes + g * 16], vals, mask=msk)
                return _

            lax.fori_loop(0, _CI // 16, grp, None, unroll=8)

        def dst(c, base, size):
            chunk = wid * cpw + c
            j = chunk // (_N // _CI)
            i0 = (chunk % (_N // _CI)) * _CI
            return out_hbm.at[pl.ds(j * _K + base, size), pl.ds(i0, _CI)]

        def chunk_body(c, _):
            for buf, sem, base, size in halves:
                @pl.when(c > 0)
                def _drain():
                    pltpu.make_async_copy(
                        buf, dst(c - 1, base, size), sem).wait()
                    scatter(buf, c - 1, base, size, zeros)

                scatter(buf, c, base, size, ones)
                pltpu.make_async_copy(buf, dst(c, base, size), sem).start()
            return _

        lax.fori_loop(0, cpw, chunk_body, None)
        for buf, sem, base, size in halves:
            pltpu.make_async_copy(buf, dst(cpw - 1, base, size), sem).wait()

    return sc_onehot


def kernel(inputs):
    idx_flat = inputs.T.reshape(-1)
    out2d = _make_sc_kernel()(idx_flat)
    out_t = out2d.reshape(_M, _K, _N)
    return jnp.transpose(out_t, (2, 0, 1))


# speedup vs baseline: 1.0094x; 1.0094x over previous
import functools

import jax
import jax.numpy as jnp
from jax import lax
from jax.experimental import pallas as pl
from jax.experimental.pallas import tpu as pltpu
from jax.experimental.pallas import tpu_sc as plsc

_N = 4096
_M = 20
_K = 1000
_CI = 128
_NCHUNK = _M * (_N // _CI)
_H0 = 512
_H1 = _K - _H0


def _make_sc_kernel():
    info = plsc.get_sparse_core_info()
    nc, ns = info.num_cores, info.num_subcores
    nw = nc * ns
    cpw = _NCHUNK // nw
    mesh = plsc.VectorSubcoreMesh(core_axis_name="c", subcore_axis_name="s")

    @functools.partial(
        pl.kernel,
        mesh=mesh,
        compiler_params=pltpu.CompilerParams(needs_layout_passes=False),
        out_type=jax.ShapeDtypeStruct((_M * _K, _N), jnp.float32),
        scratch_types=[
            pltpu.VMEM((_H0, _CI), jnp.float32),
            pltpu.VMEM((_H1, _CI), jnp.float32),
            pltpu.VMEM((_NCHUNK // 32 * _CI,), jnp.int32),
            pltpu.SemaphoreType.DMA,
            pltpu.SemaphoreType.DMA,
        ],
    )
    def sc_onehot(idx_hbm, out_hbm, buf0, buf1, idx_v, sem0, sem1):
        wid = lax.axis_index("s") * nc + lax.axis_index("c")
        lanes = lax.iota(jnp.int32, 16)
        ones = jnp.ones((16,), jnp.float32)
        zeros = jnp.zeros((16,), jnp.float32)
        halves = ((buf0, sem0, 0, _H0), (buf1, sem1, _H0, _H1))

        pltpu.sync_copy(idx_hbm.at[pl.ds(wid * cpw * _CI, cpw * _CI)], idx_v)

        for _buf, _sz in ((buf0, _H0), (buf1, _H1)):
            def zero_body(z, _, _buf=_buf):
                cell = lanes + z * 16
                plsc.store_scatter(_buf, [cell // _CI, cell % _CI], zeros)
                return _

            lax.fori_loop(0, _sz * _CI // 16, zero_body, None, unroll=8)

        def scatter(buf, c, base, size, vals):
            def grp(g, _):
                ids = idx_v[pl.ds(c * _CI + g * 16, 16)]
                rel = ids - base
                msk = jnp.logical_and(rel >= 0, rel < size)
                rel = jnp.where(msk, rel, 0)
                plsc.store_scatter(
                    buf, [rel, lanes + g * 16], vals, mask=msk)
                return _

            lax.fori_loop(0, _CI // 16, grp, None, unroll=8)

        def dst(c, base, size):
            chunk = wid * cpw + c
            j = chunk // (_N // _CI)
            i0 = (chunk % (_N // _CI)) * _CI
            return out_hbm.at[pl.ds(j * _K + base, size), pl.ds(i0, _CI)]

        def chunk_body(c, _):
            for buf, sem, base, size in halves:
                @pl.when(c > 0)
                def _drain():
                    pltpu.make_async_copy(
                        buf, dst(c - 1, base, size), sem).wait()
                    scatter(buf, c - 1, base, size, zeros)

                scatter(buf, c, base, size, ones)
                pltpu.make_async_copy(buf, dst(c, base, size), sem).start()
            return _

        lax.fori_loop(0, cpw, chunk_body, None)
        for buf, sem, base, size in halves:
            pltpu.make_async_copy(buf, dst(cpw - 1, base, size), sem).wait()

    return sc_onehot


def kernel(inputs):
    idx_flat = inputs.T.reshape(-1)
    out2d = _make_sc_kernel()(idx_flat)
    out_t = out2d.reshape(_M, _K, _N)
    return jnp.transpose(out_t, (2, 0, 1))
